# Initial kernel scaffold; baseline (speedup 1.0000x reference)
import jax
import jax.numpy as jnp
from jax.experimental import pallas as pl


def kernel(x, edge_index, W1, b1, W2, b2):
    raise NotImplementedError("write your pallas kernel here")


# speedup vs baseline: 13.6018x; 13.6018x over previous
import functools

import jax
import jax.numpy as jnp
from jax import lax
from jax.experimental import pallas as pl
from jax.experimental.pallas import tpu as pltpu
from jax.experimental.pallas import tpu_sc as plsc

NC = 2
NS = 16
TILES = NC * NS
CHUNK = 128


def _mesh():
    return plsc.VectorSubcoreMesh(
        core_axis_name="c", subcore_axis_name="s",
        num_cores=NC, num_subcores=NS)


def _sc_degree(dstp, npad):
    _, ch, _ = dstp.shape
    rp = npad // NS

    @functools.partial(
        pl.kernel,
        out_type=jax.ShapeDtypeStruct((NC, npad), jnp.float32),
        mesh=_mesh(),
        scratch_types=[
            pltpu.VMEM((ch, CHUNK), jnp.int32),
            pltpu.VMEM((CHUNK,), jnp.float32),
            pltpu.VMEM((CHUNK,), jnp.float32),
            pltpu.VMEM_SHARED((npad,), jnp.float32),
        ],
    )
    def k(dst_hbm, out_hbm, idx_v, ones_v, zero_v, deg_sh):
        c = lax.axis_index("c")
        s = lax.axis_index("s")
        wid = c * NS + s
        one = jnp.full((16,), 1.0, jnp.float32)
        zero = jnp.zeros((16,), jnp.float32)
        for q in range(CHUNK // 16):
            ones_v[pl.ds(q * 16, 16)] = one
            zero_v[pl.ds(q * 16, 16)] = zero
        @pl.loop(0, rp // CHUNK)
        def _zero(j):
            pltpu.sync_copy(zero_v, deg_sh.at[pl.ds(s * rp + j * CHUNK, CHUNK)])
        pltpu.sync_copy(dst_hbm.at[wid], idx_v)
        plsc.subcore_barrier()

        @pl.loop(0, ch)
        def _acc(j):
            pltpu.sync_copy(ones_v, deg_sh.at[idx_v.at[j]], add=True)

        plsc.subcore_barrier()
        pltpu.sync_copy(deg_sh.at[pl.ds(s * rp, rp)],
                        out_hbm.at[c, pl.ds(s * rp, rp)])

    return k(dstp)


def _sc_scatter(srcp, dstp, h, npad):
    _, ch, _ = srcp.shape
    d = h.shape[1]
    rp = npad // NS

    @functools.partial(
        pl.kernel,
        out_type=jax.ShapeDtypeStruct((NC, npad, d), jnp.float32),
        mesh=_mesh(),
        scratch_types=[
            pltpu.VMEM((ch, CHUNK), jnp.int32),
            pltpu.VMEM((ch, CHUNK), jnp.int32),
            pltpu.VMEM((CHUNK, d), jnp.float32),
            pltpu.VMEM((8, d), jnp.float32),
            pltpu.VMEM_SHARED((npad, d), jnp.float32),
            pltpu.SemaphoreType.DMA,
        ],
    )
    def k(src_hbm, dst_hbm, h_hbm, out_hbm,
          src_v, dst_v, rows_v, zrow_v, acc_sh, sem):
        c = lax.axis_index("c")
        s = lax.axis_index("s")
        wid = c * NS + s
        zero = jnp.zeros((16,), jnp.float32)
        for r in range(8):
            for q in range(d // 16):
                zrow_v[r, pl.ds(q * 16, 16)] = zero
        @pl.loop(0, rp // 8)
        def _zero(j):
            pltpu.sync_copy(zrow_v, acc_sh.at[pl.ds(s * rp + j * 8, 8)])
        pltpu.sync_copy(src_hbm.at[wid], src_v)
        pltpu.sync_copy(dst_hbm.at[wid], dst_v)
        plsc.subcore_barrier()

        @pl.loop(0, ch)
        def _edges(j):
            pltpu.async_copy(h_hbm.at[src_v.at[j]], rows_v, sem).wait()
            pltpu.sync_copy(rows_v, acc_sh.at[dst_v.at[j]], add=True)

        plsc.subcore_barrier()
        pltpu.sync_copy(acc_sh.at[pl.ds(s * rp, rp)],
                        out_hbm.at[c, pl.ds(s * rp, rp)])

    return k(srcp, dstp, h)


def _dinv_of(deg_ref):
    dtot = deg_ref[:, 0:1] + deg_ref[:, 1:2] + 1.0
    return lax.rsqrt(dtot)


def _tc_matmul_scale(x, w, deg2t):
    n, d = x.shape
    r = 1000

    def body(x_ref, w_ref, deg_ref, o_ref):
        mm = jnp.dot(x_ref[...], w_ref[...], preferred_element_type=jnp.float32)
        o_ref[...] = mm * _dinv_of(deg_ref)

    return pl.pallas_call(
        body,
        grid=(n // r,),
        in_specs=[pl.BlockSpec((r, d), lambda i: (i, 0)),
                  pl.BlockSpec((d, d), lambda i: (0, 0)),
                  pl.BlockSpec((r, 2), lambda i: (i, 0))],
        out_specs=pl.BlockSpec((r, d), lambda i: (i, 0)),
        out_shape=jax.ShapeDtypeStruct((n, d), jnp.float32),
    )(x, w, deg2t)


def _tc_combine_matmul(acc, hp, b, deg2t, w):
    n, d = hp.shape
    r = 1000

    def body(a_ref, hp_ref, b_ref, deg_ref, w_ref, o_ref):
        dinv = _dinv_of(deg_ref)
        agg = a_ref[0] + a_ref[1] + hp_ref[...]
        y = jnp.maximum(agg * dinv + b_ref[...], 0.0)
        o_ref[...] = jnp.dot(y, w_ref[...],
                             preferred_element_type=jnp.float32) * dinv

    return pl.pallas_call(
        body,
        grid=(n // r,),
        in_specs=[pl.BlockSpec((NC, r, d), lambda i: (0, i, 0)),
                  pl.BlockSpec((r, d), lambda i: (i, 0)),
                  pl.BlockSpec((1, d), lambda i: (0, 0)),
                  pl.BlockSpec((r, 2), lambda i: (i, 0)),
                  pl.BlockSpec((d, d), lambda i: (0, 0))],
        out_specs=pl.BlockSpec((r, d), lambda i: (i, 0)),
        out_shape=jax.ShapeDtypeStruct((n, d), jnp.float32),
    )(acc, hp, b, deg2t, w)


def _tc_final(acc, hp, b, deg2t):
    n, d = hp.shape
    r = 1000

    def body(a_ref, hp_ref, b_ref, deg_ref, o_ref):
        agg = a_ref[0] + a_ref[1] + hp_ref[...]
        o_ref[...] = jnp.maximum(agg * _dinv_of(deg_ref) + b_ref[...], 0.0)

    return pl.pallas_call(
        body,
        grid=(n // r,),
        in_specs=[pl.BlockSpec((NC, r, d), lambda i: (0, i, 0)),
                  pl.BlockSpec((r, d), lambda i: (i, 0)),
                  pl.BlockSpec((1, d), lambda i: (0, 0)),
                  pl.BlockSpec((r, 2), lambda i: (i, 0))],
        out_specs=pl.BlockSpec((r, d), lambda i: (i, 0)),
        out_shape=jax.ShapeDtypeStruct((n, d), jnp.float32),
    )(acc, hp, b, deg2t)


def kernel(x, edge_index, W1, b1, W2, b2):
    n, d = x.shape
    e = edge_index.shape[1]
    npad = ((n + 1023) // 1024) * 1024
    ch = -(-e // (TILES * CHUNK))
    epad = TILES * ch * CHUNK

    src = edge_index[0]
    dst = edge_index[1]
    srcp = jnp.concatenate(
        [src, jnp.zeros((epad - e,), jnp.int32)]).reshape(TILES, ch, CHUNK)
    dstp = jnp.concatenate(
        [dst, jnp.full((epad - e,), n, jnp.int32)]).reshape(TILES, ch, CHUNK)

    deg2 = _sc_degree(dstp, npad)
    deg2t = deg2.T[:n]
    b1r = b1.reshape(1, d)
    b2r = b2.reshape(1, d)

    h1 = _tc_matmul_scale(x, W1, deg2t)
    acc1 = _sc_scatter(srcp, dstp, h1, npad)
    h2 = _tc_combine_matmul(acc1, h1, b1r, deg2t, W2)
    acc2 = _sc_scatter(srcp, dstp, h2, npad)
    return _tc_final(acc2, h2, b2r, deg2t)
